# baseline (speedup 1.0000x reference)
import jax
import jax.numpy as jnp
from jax import lax
from jax.experimental import pallas as pl
from jax.experimental.pallas import tpu as pltpu
from jax.experimental.pallas import tpu_sc as plsc

VOCAB = 1000000
DIM = 64
BATCH = 16384

_info = plsc.get_sparse_core_info()
_NC, _NS = _info.num_cores, _info.num_subcores
_NW = _NC * _NS
_B_PER_W = BATCH // _NW
_TBLK = 32768
_NBLK = (VOCAB + _TBLK - 1) // _TBLK
_TROWS = _NBLK * _TBLK


def _transpose_body(x_ref, i_ref, o_ref):
    xt = lax.dot_general(
        x_ref[...], i_ref[...], (((0,), (0,)), ((), ())),
        preferred_element_type=jnp.float32,
    )
    o_ref[...] = jnp.concatenate(
        [xt[: _TBLK // 2, :], xt[_TBLK // 2 :, :]], axis=1
    )


@jax.jit
def _tc_relayout(table_t, eye):
    return pl.pallas_call(
        _transpose_body,
        grid=(_NBLK,),
        in_specs=[
            pl.BlockSpec((DIM, _TBLK), lambda i: (0, i)),
            pl.BlockSpec((DIM, DIM), lambda i: (0, 0)),
        ],
        out_specs=pl.BlockSpec((_TBLK // 2, 2 * DIM), lambda i: (i, 0)),
        out_shape=jax.ShapeDtypeStruct((_TROWS // 2, 2 * DIM), jnp.float32),
    )(table_t, eye)

_ROUNDS = 16
_RCHUNK = _B_PER_W // _ROUNDS


def _gather_body(idx_hbm, t_hbm, out_hbm, idx_v, blk_v, rows_v, sem):
    wid = lax.axis_index("s") * _NC + lax.axis_index("c")
    base = wid * _B_PER_W
    pltpu.sync_copy(idx_hbm.at[pl.ds(base, _B_PER_W)], idx_v)

    def per_round(c, _):
        for i in range(_RCHUNK // 16):
            ivec = idx_v[pl.ds(c * _RCHUNK + i * 16, 16)]
            uvec = ((ivec >> 15) << 14) | (ivec & 16383)
            q8vec = (uvec >> 3) << 3
            for j in range(16):
                q8 = pl.multiple_of(
                    lax.squeeze(lax.slice(q8vec, (j,), (j + 1,)), (0,)), 8
                )
                g = i * 16 + j
                pltpu.async_copy(
                    t_hbm.at[pl.ds(q8, 8), :],
                    blk_v.at[pl.ds(g * 8, 8), :],
                    sem,
                )
        pltpu.make_async_copy(
            t_hbm.at[pl.ds(0, 8 * _RCHUNK), :], blk_v, sem
        ).wait()
        for i in range(_RCHUNK // 16):
            ivec = idx_v[pl.ds(c * _RCHUNK + i * 16, 16)]
            svec = (ivec & 7) | (((ivec >> 14) & 1) << 9)
            for j in range(16):
                g = i * 16 + j
                sj = lax.squeeze(lax.slice(svec, (j,), (j + 1,)), (0,))
                row = (sj & 7) + g * 8
                off = (sj >> 9) * DIM
                for k in range(4):
                    rows_v[g, pl.ds(k * 16, 16)] = blk_v[
                        row, pl.ds(off + k * 16, 16)
                    ]
        pltpu.sync_copy(
            rows_v, out_hbm.at[pl.ds(base + c * _RCHUNK, _RCHUNK), :]
        )
        return 0

    lax.fori_loop(0, _ROUNDS, per_round, 0)


@jax.jit
def _sc_gather(indices, t):
    mesh = plsc.VectorSubcoreMesh(core_axis_name="c", subcore_axis_name="s")
    return pl.kernel(
        _gather_body,
        mesh=mesh,
        out_type=jax.ShapeDtypeStruct((BATCH, DIM), jnp.float32),
        scratch_types=[
            pltpu.VMEM((_B_PER_W,), jnp.int32),
            pltpu.VMEM((8 * _RCHUNK, 2 * DIM), jnp.float32),
            pltpu.VMEM((_RCHUNK, DIM), jnp.float32),
            pltpu.SemaphoreType.DMA,
        ],
    )(indices, t)


_BLK = 2048


def _mlp_body(x_ref, w1_ref, b1_ref, w2_ref, b2_ref, o_ref):
    x = jnp.maximum(x_ref[...], 0.0)
    h = lax.dot_general(
        x, w1_ref[...], (((1,), (1,)), ((), ())),
        preferred_element_type=jnp.float32,
    )
    h = jnp.maximum(h + b1_ref[...], 0.0)
    o = lax.dot_general(
        h, w2_ref[...], (((1,), (1,)), ((), ())),
        preferred_element_type=jnp.float32,
    )
    o_ref[...] = jnp.maximum(o + b2_ref[...], 0.0)


@jax.jit
def _tc_mlp(x, W1, b1, W2, b2):
    grid = (BATCH // _BLK,)
    return pl.pallas_call(
        _mlp_body,
        grid=grid,
        in_specs=[
            pl.BlockSpec((_BLK, DIM), lambda i: (i, 0)),
            pl.BlockSpec((DIM, DIM), lambda i: (0, 0)),
            pl.BlockSpec((1, DIM), lambda i: (0, 0)),
            pl.BlockSpec((DIM, DIM), lambda i: (0, 0)),
            pl.BlockSpec((1, DIM), lambda i: (0, 0)),
        ],
        out_specs=pl.BlockSpec((_BLK, DIM), lambda i: (i, 0)),
        out_shape=jax.ShapeDtypeStruct((BATCH, DIM), jnp.float32),
    )(x, W1, b1, W2, b2)


def kernel(indices, offsets, table, W1, b1, W2, b2):
    del offsets
    idx = jnp.asarray(indices, jnp.int32)
    t = _tc_relayout(table.T, jnp.eye(DIM, dtype=jnp.float32))
    x = _sc_gather(idx, t)
    return _tc_mlp(x, W1, b1.reshape(1, DIM), W2, b2.reshape(1, DIM))
